# Initial kernel scaffold; baseline (speedup 1.0000x reference)
import jax
import jax.numpy as jnp
from jax.experimental import pallas as pl


def kernel(offsets, keys, values, id2index):
    raise NotImplementedError("write your pallas kernel here")


# speedup vs baseline: 49.1251x; 49.1251x over previous
import functools

import jax
import jax.numpy as jnp
from jax import lax
from jax.experimental import pallas as pl
from jax.experimental.pallas import tpu as pltpu
from jax.experimental.pallas import tpu_sc as plsc

_NUM_CORES = 2
_NUM_SUBCORES = 16
_NUM_WORKERS = _NUM_CORES * _NUM_SUBCORES


def _remap_body(b_per_w, keys_hbm, table_hbm, out_hbm, idx_v, rows_v, sem):
    wid = lax.axis_index("s") * _NUM_CORES + lax.axis_index("c")
    base = wid * b_per_w
    pltpu.sync_copy(keys_hbm.at[pl.ds(base, b_per_w)], idx_v)
    pltpu.async_copy(table_hbm.at[idx_v], rows_v, sem).wait()
    pltpu.sync_copy(rows_v, out_hbm.at[pl.ds(base, b_per_w)])


def kernel(offsets, keys, values, id2index):
    total = keys.shape[0]
    b_per_w = total // _NUM_WORKERS
    mesh = plsc.VectorSubcoreMesh(core_axis_name="c", subcore_axis_name="s")
    remap = pl.kernel(
        functools.partial(_remap_body, b_per_w),
        mesh=mesh,
        out_type=jax.ShapeDtypeStruct((total,), jnp.int32),
        scratch_types=[
            pltpu.VMEM((b_per_w,), jnp.int32),
            pltpu.VMEM((b_per_w,), jnp.int32),
            pltpu.SemaphoreType.DMA,
        ],
    )
    idx_keys = remap(keys, id2index)
    return (offsets, idx_keys, values)


# speedup vs baseline: 59.6645x; 1.2145x over previous
import functools

import jax
import jax.numpy as jnp
from jax import lax
from jax.experimental import pallas as pl
from jax.experimental.pallas import tpu as pltpu
from jax.experimental.pallas import tpu_sc as plsc

_NUM_CORES = 2
_NUM_SUBCORES = 16
_NUM_WORKERS = _NUM_CORES * _NUM_SUBCORES


def _remap_body(b_per_w, keys_hbm, table_hbm, out_hbm, idx_v, rows_v, tab_sh, sem):
    s = lax.axis_index("s")
    wid = s * _NUM_CORES + lax.axis_index("c")
    base = wid * b_per_w
    pltpu.sync_copy(keys_hbm.at[pl.ds(base, b_per_w)], idx_v)

    @pl.when(s == 0)
    def _stage_table():
        pltpu.sync_copy(table_hbm, tab_sh)

    plsc.subcore_barrier()
    pltpu.async_copy(tab_sh.at[idx_v], rows_v, sem).wait()
    pltpu.sync_copy(rows_v, out_hbm.at[pl.ds(base, b_per_w)])


def kernel(offsets, keys, values, id2index):
    total = keys.shape[0]
    b_per_w = total // _NUM_WORKERS
    mesh = plsc.VectorSubcoreMesh(core_axis_name="c", subcore_axis_name="s")
    remap = pl.kernel(
        functools.partial(_remap_body, b_per_w),
        mesh=mesh,
        out_type=jax.ShapeDtypeStruct((total,), jnp.int32),
        scratch_types=[
            pltpu.VMEM((b_per_w,), jnp.int32),
            pltpu.VMEM((b_per_w,), jnp.int32),
            pltpu.VMEM_SHARED((id2index.shape[0],), jnp.int32),
            pltpu.SemaphoreType.DMA,
        ],
    )
    idx_keys = remap(keys, id2index)
    return (offsets, idx_keys, values)
